# baseline (speedup 1.0000x reference)
import jax
import jax.numpy as jnp
from jax import lax
from jax.experimental import pallas as pl
from jax.experimental.pallas import tpu as pltpu
from jax.experimental.pallas import tpu_sc as plsc

_N = 10000
_E = 320000
_D_IN = 128
_D_H = 64
_D_Z = 16
_DW = 128

_NC = 2
_NS = 16
_NW = _NC * _NS

_NP = 10240
_RPT = _NP // _NS
_CHUNK = 128
_CPT = 80
_CPH = 40
_EPT = _CPT * _CHUNK
_EP = _EPT * _NW
_ECH = _EP // _CHUNK

_BM = 1280
_GRID = _NP // _BM

_mesh = plsc.VectorSubcoreMesh(core_axis_name="c", subcore_axis_name="s")


def _deg_body(dst_hbm, zero_hbm, o0_hbm, o1_hbm, dst_v, ones_v, deg_sh):
    c = lax.axis_index("c")
    s = lax.axis_index("s")
    w = c * _NS + s
    rows = pl.ds(s * _RPT, _RPT)
    pltpu.sync_copy(zero_hbm.at[rows], deg_sh.at[rows])
    pltpu.sync_copy(dst_hbm.at[pl.ds(w * _CPT, _CPT)], dst_v)
    for k in range(_CHUNK // 16):
        ones_v[pl.ds(k * 16, 16)] = jnp.ones((16,), jnp.float32)
    plsc.subcore_barrier()

    def chunk(j, carry):
        pltpu.sync_copy(ones_v, deg_sh.at[dst_v.at[j]], add=True)
        return carry

    lax.fori_loop(0, _CPT, chunk, 0)
    plsc.subcore_barrier()

    @pl.when(c == 0)
    def _():
        pltpu.sync_copy(deg_sh.at[rows], o0_hbm.at[rows])

    @pl.when(c == 1)
    def _():
        pltpu.sync_copy(deg_sh.at[rows], o1_hbm.at[rows])


_deg_kernel = pl.kernel(
    _deg_body,
    out_type=(
        jax.ShapeDtypeStruct((_NP,), jnp.float32),
        jax.ShapeDtypeStruct((_NP,), jnp.float32),
    ),
    mesh=_mesh,
    scratch_types=[
        pltpu.VMEM((_CPT, _CHUNK), jnp.int32),
        pltpu.VMEM((_CHUNK,), jnp.float32),
        pltpu.VMEM_SHARED((_NP,), jnp.float32),
    ],
)


def _agg_body(u_hbm, src_hbm, dst_hbm, zero_hbm, o0_hbm, o1_hbm,
              src_v, dst_v, buf0_v, buf1_v, acc_sh, g0, g1, s0, s1):
    c = lax.axis_index("c")
    s = lax.axis_index("s")
    w = c * _NS + s
    rows = pl.ds(s * _RPT, _RPT)
    pltpu.sync_copy(zero_hbm.at[rows], acc_sh.at[rows])
    plsc.subcore_barrier()

    for h in range(_CPT // _CPH):
        base = w * _CPT + h * _CPH
        pltpu.sync_copy(src_hbm.at[pl.ds(base, _CPH)], src_v)
        pltpu.sync_copy(dst_hbm.at[pl.ds(base, _CPH)], dst_v)
        pltpu.async_copy(u_hbm.at[src_v.at[0]], buf0_v, g0)

        def pair(i, carry):
            j0 = 2 * i
            j1 = j0 + 1
            pltpu.make_async_copy(u_hbm.at[src_v.at[j0]], buf0_v, g0).wait()
            pltpu.async_copy(buf0_v, acc_sh.at[dst_v.at[j0]], s0, add=True)

            @pl.when(i > 0)
            def _():
                pltpu.make_async_copy(
                    buf1_v, acc_sh.at[dst_v.at[0]], s1).wait()

            pltpu.async_copy(u_hbm.at[src_v.at[j1]], buf1_v, g1)
            pltpu.make_async_copy(buf0_v, acc_sh.at[dst_v.at[0]], s0).wait()

            @pl.when(j0 + 2 < _CPH)
            def _():
                pltpu.async_copy(u_hbm.at[src_v.at[j0 + 2]], buf0_v, g0)

            pltpu.make_async_copy(u_hbm.at[src_v.at[j1]], buf1_v, g1).wait()
            pltpu.async_copy(buf1_v, acc_sh.at[dst_v.at[j1]], s1, add=True)
            return carry

        lax.fori_loop(0, _CPH // 2, pair, 0)
        pltpu.make_async_copy(buf1_v, acc_sh.at[dst_v.at[0]], s1).wait()
    plsc.subcore_barrier()

    @pl.when(c == 0)
    def _():
        pltpu.sync_copy(acc_sh.at[rows], o0_hbm.at[rows])

    @pl.when(c == 1)
    def _():
        pltpu.sync_copy(acc_sh.at[rows], o1_hbm.at[rows])


_agg_kernel = pl.kernel(
    _agg_body,
    out_type=(
        jax.ShapeDtypeStruct((_NP, _DW), jnp.float32),
        jax.ShapeDtypeStruct((_NP, _DW), jnp.float32),
    ),
    mesh=_mesh,
    scratch_types=[
        pltpu.VMEM((_CPH, _CHUNK), jnp.int32),
        pltpu.VMEM((_CPH, _CHUNK), jnp.int32),
        pltpu.VMEM((_CHUNK, _DW), jnp.float32),
        pltpu.VMEM((_CHUNK, _DW), jnp.float32),
        pltpu.VMEM_SHARED((_NP, _DW), jnp.float32),
        pltpu.SemaphoreType.DMA,
        pltpu.SemaphoreType.DMA,
        pltpu.SemaphoreType.DMA,
        pltpu.SemaphoreType.DMA,
    ],
)


def _dinv_of(d0_ref, d1_ref):
    deg = d0_ref[0, 0, :] + d1_ref[0, 0, :] + 1.0
    return lax.rsqrt(deg)


def _tc1_body(d0_ref, d1_ref, x_ref, w1_ref, u1_ref):
    dinv = _dinv_of(d0_ref, d1_ref)
    xw = jnp.dot(x_ref[...], w1_ref[...], preferred_element_type=jnp.float32)
    u1_ref[...] = xw * dinv[:, None]


_tc1 = pl.pallas_call(
    _tc1_body,
    grid=(_GRID,),
    in_specs=[
        pl.BlockSpec((1, 1, _BM), lambda i: (i, 0, 0)),
        pl.BlockSpec((1, 1, _BM), lambda i: (i, 0, 0)),
        pl.BlockSpec((_BM, _D_IN), lambda i: (i, 0)),
        pl.BlockSpec((_D_IN, _DW), lambda i: (0, 0)),
    ],
    out_specs=pl.BlockSpec((_BM, _DW), lambda i: (i, 0)),
    out_shape=jax.ShapeDtypeStruct((_NP, _DW), jnp.float32),
)


def _tc2_body(d0_ref, d1_ref, p0_ref, p1_ref, u1_ref, b1_ref, u2_ref):
    i = pl.program_id(0)
    dinv = _dinv_of(d0_ref, d1_ref)
    su = p0_ref[...] + p1_ref[...] + u1_ref[...]
    h = jnp.maximum(su * dinv[:, None] + b1_ref[...], 0.0)
    rowid = i * _BM + lax.broadcasted_iota(jnp.int32, (_BM, 1), 0)
    u2_ref[...] = jnp.where(rowid < _N, h * dinv[:, None], 0.0)


_tc2 = pl.pallas_call(
    _tc2_body,
    grid=(_GRID,),
    in_specs=[
        pl.BlockSpec((1, 1, _BM), lambda i: (i, 0, 0)),
        pl.BlockSpec((1, 1, _BM), lambda i: (i, 0, 0)),
        pl.BlockSpec((_BM, _DW), lambda i: (i, 0)),
        pl.BlockSpec((_BM, _DW), lambda i: (i, 0)),
        pl.BlockSpec((_BM, _DW), lambda i: (i, 0)),
        pl.BlockSpec((1, _DW), lambda i: (0, 0)),
    ],
    out_specs=pl.BlockSpec((_BM, _DW), lambda i: (i, 0)),
    out_shape=jax.ShapeDtypeStruct((_NP, _DW), jnp.float32),
)


def _tc3_body(d0_ref, d1_ref, q0_ref, q1_ref, u2_ref, wmu_ref, bmu_ref,
              wls_ref, bls_ref, eps_ref, wdec_ref, mdec_ref,
              mu_ref, ls_ref, z_ref, xr_ref):
    dinv = _dinv_of(d0_ref, d1_ref)
    t = (q0_ref[...] + q1_ref[...] + u2_ref[...]) * dinv[:, None]
    mu = jnp.dot(t, wmu_ref[...], preferred_element_type=jnp.float32) + bmu_ref[...]
    ls = jnp.dot(t, wls_ref[...], preferred_element_type=jnp.float32) + bls_ref[...]
    z = mu + eps_ref[...] * jnp.exp(ls)
    mu_ref[...] = mu
    ls_ref[...] = ls
    z_ref[...] = z
    wd = wdec_ref[...] * mdec_ref[...]
    xr_ref[...] = jnp.dot(z, wd, preferred_element_type=jnp.float32)


_tc3 = pl.pallas_call(
    _tc3_body,
    grid=(_GRID,),
    in_specs=[
        pl.BlockSpec((1, 1, _BM), lambda i: (i, 0, 0)),
        pl.BlockSpec((1, 1, _BM), lambda i: (i, 0, 0)),
        pl.BlockSpec((_BM, _DW), lambda i: (i, 0)),
        pl.BlockSpec((_BM, _DW), lambda i: (i, 0)),
        pl.BlockSpec((_BM, _DW), lambda i: (i, 0)),
        pl.BlockSpec((_DW, _D_Z), lambda i: (0, 0)),
        pl.BlockSpec((1, _D_Z), lambda i: (0, 0)),
        pl.BlockSpec((_DW, _D_Z), lambda i: (0, 0)),
        pl.BlockSpec((1, _D_Z), lambda i: (0, 0)),
        pl.BlockSpec((_BM, _D_Z), lambda i: (i, 0)),
        pl.BlockSpec((_D_Z, _D_IN), lambda i: (0, 0)),
        pl.BlockSpec((_D_Z, _D_IN), lambda i: (0, 0)),
    ],
    out_specs=[
        pl.BlockSpec((_BM, _D_Z), lambda i: (i, 0)),
        pl.BlockSpec((_BM, _D_Z), lambda i: (i, 0)),
        pl.BlockSpec((_BM, _D_Z), lambda i: (i, 0)),
        pl.BlockSpec((_BM, _D_IN), lambda i: (i, 0)),
    ],
    out_shape=[
        jax.ShapeDtypeStruct((_NP, _D_Z), jnp.float32),
        jax.ShapeDtypeStruct((_NP, _D_Z), jnp.float32),
        jax.ShapeDtypeStruct((_NP, _D_Z), jnp.float32),
        jax.ShapeDtypeStruct((_NP, _D_IN), jnp.float32),
    ],
)


def _tc4_body(zi_ref, zj_ref, adj_ref):
    adj_ref[...] = lax.dot_general(
        zi_ref[...], zj_ref[...], (((1,), (1,)), ((), ())),
        preferred_element_type=jnp.float32)


_tc4 = pl.pallas_call(
    _tc4_body,
    grid=(_GRID, _GRID),
    in_specs=[
        pl.BlockSpec((_BM, _D_Z), lambda i, j: (i, 0)),
        pl.BlockSpec((_BM, _D_Z), lambda i, j: (j, 0)),
    ],
    out_specs=pl.BlockSpec((_BM, _BM), lambda i, j: (i, j)),
    out_shape=jax.ShapeDtypeStruct((_N, _N), jnp.float32),
)


def kernel(x, edge_index, W1, b1, W_mu, b_mu, W_ls, b_ls, W_dec, mask, eps):
    src = edge_index[0]
    dst = edge_index[1]
    pad_ids = _N + (jnp.arange(_EP - _E, dtype=jnp.int32) % 16)
    srcp = jnp.concatenate([src, pad_ids]).reshape(_ECH, _CHUNK)
    dstp = jnp.concatenate([dst, pad_ids]).reshape(_ECH, _CHUNK)
    xp = jnp.pad(x, ((0, _NP - _N), (0, 0)))
    epsp = jnp.pad(eps, ((0, _NP - _N), (0, 0)))
    w1p = jnp.pad(W1, ((0, 0), (0, _DW - _D_H)))
    b1p = jnp.pad(b1, (0, _DW - _D_H)).reshape(1, _DW)
    wmup = jnp.pad(W_mu, ((0, _DW - _D_H), (0, 0)))
    wlsp = jnp.pad(W_ls, ((0, _DW - _D_H), (0, 0)))
    zeros_n = jnp.zeros((_NP,), jnp.float32)
    zeros_nw = jnp.zeros((_NP, _DW), jnp.float32)

    d0, d1 = _deg_kernel(dstp, zeros_n)
    d0 = d0.reshape(_GRID, 1, _BM)
    d1 = d1.reshape(_GRID, 1, _BM)

    u1 = _tc1(d0, d1, xp, w1p)
    p0, p1 = _agg_kernel(u1, srcp, dstp, zeros_nw)
    u2 = _tc2(d0, d1, p0, p1, u1, b1p)
    q0, q1 = _agg_kernel(u2, srcp, dstp, zeros_nw)
    mu, ls, z, xr = _tc3(d0, d1, q0, q1, u2, wmup, b_mu.reshape(1, _D_Z),
                         wlsp, b_ls.reshape(1, _D_Z), epsp, W_dec, mask)
    adj = _tc4(z, z)
    return adj, xr[:_N], mu[:_N], ls[:_N]


# speedup vs baseline: 1.1369x; 1.1369x over previous
import jax
import jax.numpy as jnp
from jax import lax
from jax.experimental import pallas as pl
from jax.experimental.pallas import tpu as pltpu
from jax.experimental.pallas import tpu_sc as plsc

_N = 10000
_E = 320000
_D_IN = 128
_D_H = 64
_D_Z = 16
_DW = 128

_NC = 2
_NS = 16
_NW = _NC * _NS

_NP = 10240
_RPT = _NP // _NS
_CHUNK = 128
_CPT = 80
_CPH = 40
_EPT = _CPT * _CHUNK
_EP = _EPT * _NW
_ECH = _EP // _CHUNK

_BM = 1280
_GRID = _NP // _BM

_mesh = plsc.VectorSubcoreMesh(core_axis_name="c", subcore_axis_name="s")


def _deg_body(dst_hbm, zero_hbm, o0_hbm, o1_hbm, dst_v, ones_v, deg_sh):
    c = lax.axis_index("c")
    s = lax.axis_index("s")
    w = c * _NS + s
    rows = pl.ds(s * _RPT, _RPT)
    pltpu.sync_copy(zero_hbm.at[rows], deg_sh.at[rows])
    pltpu.sync_copy(dst_hbm.at[pl.ds(w * _CPT, _CPT)], dst_v)
    for k in range(_CHUNK // 16):
        ones_v[pl.ds(k * 16, 16)] = jnp.ones((16,), jnp.float32)
    plsc.subcore_barrier()

    def chunk(j, carry):
        pltpu.sync_copy(ones_v, deg_sh.at[dst_v.at[j]], add=True)
        return carry

    lax.fori_loop(0, _CPT, chunk, 0)
    plsc.subcore_barrier()

    @pl.when(c == 0)
    def _():
        pltpu.sync_copy(deg_sh.at[rows], o0_hbm.at[rows])

    @pl.when(c == 1)
    def _():
        pltpu.sync_copy(deg_sh.at[rows], o1_hbm.at[rows])


_deg_kernel = pl.kernel(
    _deg_body,
    out_type=(
        jax.ShapeDtypeStruct((_NP,), jnp.float32),
        jax.ShapeDtypeStruct((_NP,), jnp.float32),
    ),
    mesh=_mesh,
    scratch_types=[
        pltpu.VMEM((_CPT, _CHUNK), jnp.int32),
        pltpu.VMEM((_CHUNK,), jnp.float32),
        pltpu.VMEM_SHARED((_NP,), jnp.float32),
    ],
)


def _agg_body(u_hbm, src_hbm, dst_hbm, zero_hbm, o0_hbm, o1_hbm,
              src_v, dst_v, buf0_v, buf1_v, acc_sh, g0, g1):
    c = lax.axis_index("c")
    s = lax.axis_index("s")
    w = c * _NS + s
    rows = pl.ds(s * _RPT, _RPT)
    pltpu.sync_copy(zero_hbm.at[rows], acc_sh.at[rows])
    plsc.subcore_barrier()

    for h in range(_CPT // _CPH):
        base = w * _CPT + h * _CPH
        pltpu.sync_copy(src_hbm.at[pl.ds(base, _CPH)], src_v)
        pltpu.sync_copy(dst_hbm.at[pl.ds(base, _CPH)], dst_v)
        pltpu.async_copy(u_hbm.at[src_v.at[0]], buf0_v, g0)

        def pair(i, carry):
            j0 = 2 * i
            j1 = j0 + 1
            pltpu.async_copy(u_hbm.at[src_v.at[j1]], buf1_v, g1)
            pltpu.make_async_copy(u_hbm.at[src_v.at[j0]], buf0_v, g0).wait()
            pltpu.sync_copy(buf0_v, acc_sh.at[dst_v.at[j0]], add=True)

            @pl.when(j0 + 2 < _CPH)
            def _():
                pltpu.async_copy(u_hbm.at[src_v.at[j0 + 2]], buf0_v, g0)

            pltpu.make_async_copy(u_hbm.at[src_v.at[j1]], buf1_v, g1).wait()
            pltpu.sync_copy(buf1_v, acc_sh.at[dst_v.at[j1]], add=True)
            return carry

        lax.fori_loop(0, _CPH // 2, pair, 0)
    plsc.subcore_barrier()

    @pl.when(c == 0)
    def _():
        pltpu.sync_copy(acc_sh.at[rows], o0_hbm.at[rows])

    @pl.when(c == 1)
    def _():
        pltpu.sync_copy(acc_sh.at[rows], o1_hbm.at[rows])


_agg_kernel = pl.kernel(
    _agg_body,
    out_type=(
        jax.ShapeDtypeStruct((_NP, _DW), jnp.float32),
        jax.ShapeDtypeStruct((_NP, _DW), jnp.float32),
    ),
    mesh=_mesh,
    scratch_types=[
        pltpu.VMEM((_CPH, _CHUNK), jnp.int32),
        pltpu.VMEM((_CPH, _CHUNK), jnp.int32),
        pltpu.VMEM((_CHUNK, _DW), jnp.float32),
        pltpu.VMEM((_CHUNK, _DW), jnp.float32),
        pltpu.VMEM_SHARED((_NP, _DW), jnp.float32),
        pltpu.SemaphoreType.DMA,
        pltpu.SemaphoreType.DMA,
    ],
)


def _dinv_of(d0_ref, d1_ref):
    deg = d0_ref[0, 0, :] + d1_ref[0, 0, :] + 1.0
    return lax.rsqrt(deg)


def _tc1_body(d0_ref, d1_ref, x_ref, w1_ref, u1_ref):
    dinv = _dinv_of(d0_ref, d1_ref)
    xw = jnp.dot(x_ref[...], w1_ref[...], preferred_element_type=jnp.float32)
    u1_ref[...] = xw * dinv[:, None]


_tc1 = pl.pallas_call(
    _tc1_body,
    grid=(_GRID,),
    in_specs=[
        pl.BlockSpec((1, 1, _BM), lambda i: (i, 0, 0)),
        pl.BlockSpec((1, 1, _BM), lambda i: (i, 0, 0)),
        pl.BlockSpec((_BM, _D_IN), lambda i: (i, 0)),
        pl.BlockSpec((_D_IN, _DW), lambda i: (0, 0)),
    ],
    out_specs=pl.BlockSpec((_BM, _DW), lambda i: (i, 0)),
    out_shape=jax.ShapeDtypeStruct((_NP, _DW), jnp.float32),
)


def _tc2_body(d0_ref, d1_ref, p0_ref, p1_ref, u1_ref, b1_ref, u2_ref):
    i = pl.program_id(0)
    dinv = _dinv_of(d0_ref, d1_ref)
    su = p0_ref[...] + p1_ref[...] + u1_ref[...]
    h = jnp.maximum(su * dinv[:, None] + b1_ref[...], 0.0)
    rowid = i * _BM + lax.broadcasted_iota(jnp.int32, (_BM, 1), 0)
    u2_ref[...] = jnp.where(rowid < _N, h * dinv[:, None], 0.0)


_tc2 = pl.pallas_call(
    _tc2_body,
    grid=(_GRID,),
    in_specs=[
        pl.BlockSpec((1, 1, _BM), lambda i: (i, 0, 0)),
        pl.BlockSpec((1, 1, _BM), lambda i: (i, 0, 0)),
        pl.BlockSpec((_BM, _DW), lambda i: (i, 0)),
        pl.BlockSpec((_BM, _DW), lambda i: (i, 0)),
        pl.BlockSpec((_BM, _DW), lambda i: (i, 0)),
        pl.BlockSpec((1, _DW), lambda i: (0, 0)),
    ],
    out_specs=pl.BlockSpec((_BM, _DW), lambda i: (i, 0)),
    out_shape=jax.ShapeDtypeStruct((_NP, _DW), jnp.float32),
)


def _tc3_body(d0_ref, d1_ref, q0_ref, q1_ref, u2_ref, wmu_ref, bmu_ref,
              wls_ref, bls_ref, eps_ref, wdec_ref, mdec_ref,
              mu_ref, ls_ref, z_ref, xr_ref):
    dinv = _dinv_of(d0_ref, d1_ref)
    t = (q0_ref[...] + q1_ref[...] + u2_ref[...]) * dinv[:, None]
    mu = jnp.dot(t, wmu_ref[...], preferred_element_type=jnp.float32) + bmu_ref[...]
    ls = jnp.dot(t, wls_ref[...], preferred_element_type=jnp.float32) + bls_ref[...]
    z = mu + eps_ref[...] * jnp.exp(ls)
    mu_ref[...] = mu
    ls_ref[...] = ls
    z_ref[...] = z
    wd = wdec_ref[...] * mdec_ref[...]
    xr_ref[...] = jnp.dot(z, wd, preferred_element_type=jnp.float32)


_tc3 = pl.pallas_call(
    _tc3_body,
    grid=(_GRID,),
    in_specs=[
        pl.BlockSpec((1, 1, _BM), lambda i: (i, 0, 0)),
        pl.BlockSpec((1, 1, _BM), lambda i: (i, 0, 0)),
        pl.BlockSpec((_BM, _DW), lambda i: (i, 0)),
        pl.BlockSpec((_BM, _DW), lambda i: (i, 0)),
        pl.BlockSpec((_BM, _DW), lambda i: (i, 0)),
        pl.BlockSpec((_DW, _D_Z), lambda i: (0, 0)),
        pl.BlockSpec((1, _D_Z), lambda i: (0, 0)),
        pl.BlockSpec((_DW, _D_Z), lambda i: (0, 0)),
        pl.BlockSpec((1, _D_Z), lambda i: (0, 0)),
        pl.BlockSpec((_BM, _D_Z), lambda i: (i, 0)),
        pl.BlockSpec((_D_Z, _D_IN), lambda i: (0, 0)),
        pl.BlockSpec((_D_Z, _D_IN), lambda i: (0, 0)),
    ],
    out_specs=[
        pl.BlockSpec((_BM, _D_Z), lambda i: (i, 0)),
        pl.BlockSpec((_BM, _D_Z), lambda i: (i, 0)),
        pl.BlockSpec((_BM, _D_Z), lambda i: (i, 0)),
        pl.BlockSpec((_BM, _D_IN), lambda i: (i, 0)),
    ],
    out_shape=[
        jax.ShapeDtypeStruct((_NP, _D_Z), jnp.float32),
        jax.ShapeDtypeStruct((_NP, _D_Z), jnp.float32),
        jax.ShapeDtypeStruct((_NP, _D_Z), jnp.float32),
        jax.ShapeDtypeStruct((_NP, _D_IN), jnp.float32),
    ],
)


def _tc4_body(zi_ref, zj_ref, adj_ref):
    adj_ref[...] = lax.dot_general(
        zi_ref[...], zj_ref[...], (((1,), (1,)), ((), ())),
        preferred_element_type=jnp.float32)


_tc4 = pl.pallas_call(
    _tc4_body,
    grid=(_GRID, _GRID),
    in_specs=[
        pl.BlockSpec((_BM, _D_Z), lambda i, j: (i, 0)),
        pl.BlockSpec((_BM, _D_Z), lambda i, j: (j, 0)),
    ],
    out_specs=pl.BlockSpec((_BM, _BM), lambda i, j: (i, j)),
    out_shape=jax.ShapeDtypeStruct((_N, _N), jnp.float32),
)


def kernel(x, edge_index, W1, b1, W_mu, b_mu, W_ls, b_ls, W_dec, mask, eps):
    src = edge_index[0]
    dst = edge_index[1]
    pad_ids = _N + (jnp.arange(_EP - _E, dtype=jnp.int32) % 16)
    srcp = jnp.concatenate([src, pad_ids]).reshape(_ECH, _CHUNK)
    dstp = jnp.concatenate([dst, pad_ids]).reshape(_ECH, _CHUNK)
    xp = jnp.pad(x, ((0, _NP - _N), (0, 0)))
    epsp = jnp.pad(eps, ((0, _NP - _N), (0, 0)))
    w1p = jnp.pad(W1, ((0, 0), (0, _DW - _D_H)))
    b1p = jnp.pad(b1, (0, _DW - _D_H)).reshape(1, _DW)
    wmup = jnp.pad(W_mu, ((0, _DW - _D_H), (0, 0)))
    wlsp = jnp.pad(W_ls, ((0, _DW - _D_H), (0, 0)))
    zeros_n = jnp.zeros((_NP,), jnp.float32)
    zeros_nw = jnp.zeros((_NP, _DW), jnp.float32)

    d0, d1 = _deg_kernel(dstp, zeros_n)
    d0 = d0.reshape(_GRID, 1, _BM)
    d1 = d1.reshape(_GRID, 1, _BM)

    u1 = _tc1(d0, d1, xp, w1p)
    p0, p1 = _agg_kernel(u1, srcp, dstp, zeros_nw)
    u2 = _tc2(d0, d1, p0, p1, u1, b1p)
    q0, q1 = _agg_kernel(u2, srcp, dstp, zeros_nw)
    mu, ls, z, xr = _tc3(d0, d1, q0, q1, u2, wmup, b_mu.reshape(1, _D_Z),
                         wlsp, b_ls.reshape(1, _D_Z), epsp, W_dec, mask)
    adj = _tc4(z, z)
    return adj, xr[:_N], mu[:_N], ls[:_N]
